# Initial kernel scaffold; baseline (speedup 1.0000x reference)
import functools

import jax
import jax.numpy as jnp
from jax import lax
from jax.experimental import pallas as pl
from jax.experimental.pallas import tpu as pltpu
from jax.experimental.pallas import tpu_sc as plsc

N = 10000
E = 160000
NSUB = 16
NCORE = 2
BE = 128
NB = 79
EP = NSUB * BE * NB
EPT = EP // (NCORE * NSUB)
NBE = NB * BE
ROWS_PER_TILE = N // NSUB
M_BLK = 2000
M_GRID = N // M_BLK

_mesh = plsc.VectorSubcoreMesh(core_axis_name="c", subcore_axis_name="s")


@functools.partial(
    pl.kernel,
    out_type=(
        jax.ShapeDtypeStruct((NCORE * NSUB, N), jnp.float32),
        jax.ShapeDtypeStruct((NCORE * NSUB, N), jnp.float32),
    ),
    mesh=_mesh,
    scratch_types=[
        pltpu.VMEM((EPT,), jnp.int32),
        pltpu.VMEM((EPT,), jnp.float32),
        pltpu.VMEM((EPT,), jnp.float32),
        pltpu.VMEM((N,), jnp.float32),
        pltpu.VMEM((N,), jnp.float32),
    ],
)
def _degree_sc(dst2, ew2, ones2, degp, cntp, dst_v, ew_v, on_v, dacc, cacc):
    c = lax.axis_index("c")
    s = lax.axis_index("s")
    wid = s * NCORE + c
    pltpu.sync_copy(dst2.at[wid], dst_v)
    pltpu.sync_copy(ew2.at[wid], ew_v)
    pltpu.sync_copy(ones2.at[wid], on_v)

    @pl.loop(0, N // 16)
    def _zero(i):
        sl = pl.ds(i * 16, 16)
        dacc[sl] = jnp.zeros((16,), jnp.float32)
        cacc[sl] = jnp.zeros((16,), jnp.float32)

    @pl.loop(0, EPT // 16)
    def _edges(b):
        sl = pl.ds(b * 16, 16)
        idx = dst_v[sl]
        plsc.addupdate_scatter(dacc, [idx], ew_v[sl])
        plsc.addupdate_scatter(cacc, [idx], on_v[sl])

    pltpu.sync_copy(dacc, degp.at[wid])
    pltpu.sync_copy(cacc, cntp.at[wid])


def _make_agg(nch):
    nch_sc = nch // NCORE

    @functools.partial(
        pl.kernel,
        out_type=jax.ShapeDtypeStruct((nch * N, 128), jnp.float32),
        mesh=_mesh,
        scratch_types=[
            pltpu.VMEM((NB, BE), jnp.int32),
            pltpu.VMEM((NB, BE), jnp.int32),
            pltpu.VMEM((NBE,), jnp.float32),
            pltpu.VMEM((BE, 128), jnp.float32),
            pltpu.VMEM((125, 128), jnp.float32),
            pltpu.VMEM_SHARED((N, 128), jnp.float32),
            pltpu.SemaphoreType.DMA,
        ],
    )
    def _agg(g2d, gidx, dst3, ewf, s_out, gidx_v, dst_v, ew_v, gbuf, zbuf,
             acc, gsem):
        c = lax.axis_index("c")
        s = lax.axis_index("s")
        pltpu.sync_copy(dst3.at[s], dst_v)
        pltpu.sync_copy(ewf.at[s], ew_v)

        @pl.loop(0, 125)
        def _zb(r):
            for j in range(8):
                zbuf[r, pl.ds(j * 16, 16)] = jnp.zeros((16,), jnp.float32)

        for k in range(nch_sc):
            chunk = c * nch_sc + k
            for p in range(5):
                pltpu.sync_copy(
                    zbuf, acc.at[pl.ds(s * ROWS_PER_TILE + p * 125, 125)])
            pltpu.sync_copy(gidx.at[chunk * NSUB + s], gidx_v)
            plsc.subcore_barrier()

            @pl.loop(0, NB)
            def _batch(b):
                pltpu.async_copy(g2d.at[gidx_v.at[b]], gbuf, gsem).wait()

                @pl.loop(0, BE)
                def _scale(e):
                    i16 = jnp.full((16,), b * BE + e, dtype=jnp.int32)
                    w16 = plsc.load_gather(ew_v, [i16])
                    for j in range(8):
                        sl = pl.ds(j * 16, 16)
                        gbuf[e, sl] = gbuf[e, sl] * w16

                pltpu.sync_copy(gbuf, acc.at[dst_v.at[b]], add=True)

            plsc.subcore_barrier()
            pltpu.sync_copy(
                acc.at[pl.ds(s * ROWS_PER_TILE, ROWS_PER_TILE)],
                s_out.at[pl.ds(chunk * N + s * ROWS_PER_TILE, ROWS_PER_TILE)])

    return _agg


_agg2 = _make_agg(2)
_agg4 = _make_agg(4)


def _prep_body(x_ref, degp_ref, cntp_ref, g1_ref, dinv_ref, a_ref):
    deg = jnp.sum(degp_ref[...], axis=1, keepdims=True) + 1.0
    cnt = jnp.sum(cntp_ref[...], axis=1, keepdims=True) + 1.0
    dinv = jnp.where(deg > 0, lax.rsqrt(deg), 0.0)
    dinv_ref[...] = dinv
    a_ref[...] = dinv / cnt
    g1_ref[...] = x_ref[...] * dinv


def _prep(x, degpT, cntpT):
    return pl.pallas_call(
        _prep_body,
        grid=(M_GRID,),
        in_specs=[
            pl.BlockSpec((M_BLK, 256), lambda m: (m, 0)),
            pl.BlockSpec((M_BLK, NCORE * NSUB), lambda m: (m, 0)),
            pl.BlockSpec((M_BLK, NCORE * NSUB), lambda m: (m, 0)),
        ],
        out_specs=[
            pl.BlockSpec((M_BLK, 256), lambda m: (m, 0)),
            pl.BlockSpec((M_BLK, 1), lambda m: (m, 0)),
            pl.BlockSpec((M_BLK, 1), lambda m: (m, 0)),
        ],
        out_shape=[
            jax.ShapeDtypeStruct((N, 256), jnp.float32),
            jax.ShapeDtypeStruct((N, 1), jnp.float32),
            jax.ShapeDtypeStruct((N, 1), jnp.float32),
        ],
    )(x, degpT, cntpT)


def _make_layer(nch, d_out, use_s, prelu, scale_out):
    def body(*refs):
        i = 0
        if use_s:
            s_ref, g_ref, a_ref = refs[0], refs[1], refs[2]
            i = 3
        else:
            g_ref = refs[0]
            i = 1
        w_ref = refs[i]
        i += 1
        if prelu:
            b_ref, al_ref = refs[i], refs[i + 1]
            i += 2
        if scale_out:
            dinv_ref = refs[i]
            i += 1
        out_ref = refs[i]
        k = pl.program_id(1)
        if use_s:
            A = (s_ref[0] + g_ref[...]) * a_ref[...]
        else:
            A = g_ref[...]
        part = jnp.dot(A, w_ref[0], preferred_element_type=jnp.float32)

        @pl.when(k == 0)
        def _():
            out_ref[...] = part

        @pl.when(k > 0)
        def _():
            out_ref[...] += part

        @pl.when(k == nch - 1)
        def _():
            z = out_ref[...]
            if prelu:
                z = z + b_ref[...]
                z = jnp.where(z >= 0, z, al_ref[0, 0] * z)
            if scale_out:
                z = z * dinv_ref[...]
            out_ref[...] = z

    in_specs = []
    if use_s:
        in_specs.append(pl.BlockSpec((1, M_BLK, 128), lambda m, k: (k, m, 0)))
        in_specs.append(pl.BlockSpec((M_BLK, 128), lambda m, k: (m, k)))
        in_specs.append(pl.BlockSpec((M_BLK, 1), lambda m, k: (m, 0)))
    else:
        in_specs.append(pl.BlockSpec((M_BLK, 128), lambda m, k: (m, k)))
    in_specs.append(pl.BlockSpec((1, 128, d_out), lambda m, k: (k, 0, 0)))
    if prelu:
        in_specs.append(pl.BlockSpec((1, d_out), lambda m, k: (0, 0)))
        in_specs.append(pl.BlockSpec((1, 1), lambda m, k: (0, 0)))
    if scale_out:
        in_specs.append(pl.BlockSpec((M_BLK, 1), lambda m, k: (m, 0)))

    return pl.pallas_call(
        body,
        grid=(M_GRID, nch),
        in_specs=in_specs,
        out_specs=pl.BlockSpec((M_BLK, d_out), lambda m, k: (m, 0)),
        out_shape=jax.ShapeDtypeStruct((N, d_out), jnp.float32),
    )


def _final_body(s_ref, g_ref, a_ref, b_ref, out_ref):
    scat = jnp.concatenate([s_ref[0], s_ref[1]], axis=1)
    z = (scat + g_ref[...]) * a_ref[...] + b_ref[...]
    out_ref[...] = (jax.nn.sigmoid(z) - 0.5) * 2.0


def _final(s3, g3, a_, b3):
    return pl.pallas_call(
        _final_body,
        grid=(M_GRID,),
        in_specs=[
            pl.BlockSpec((2, M_BLK, 128), lambda m: (0, m, 0)),
            pl.BlockSpec((M_BLK, 256), lambda m: (m, 0)),
            pl.BlockSpec((M_BLK, 1), lambda m: (m, 0)),
            pl.BlockSpec((1, 256), lambda m: (0, 0)),
        ],
        out_specs=pl.BlockSpec((M_BLK, 256), lambda m: (m, 0)),
        out_shape=jax.ShapeDtypeStruct((N, 256), jnp.float32),
    )(s3, g3, a_, b3)


def kernel(x, edge_index, edge_attr, W1, b1, W2, b2, W3, b3, a1, a2):
    src = edge_index[0].astype(jnp.int32)
    dst = edge_index[1].astype(jnp.int32)
    ew = edge_attr.astype(jnp.float32)
    pad = EP - E
    srcp = jnp.pad(src, (0, pad))
    dstp = jnp.pad(dst, (0, pad))
    ewp = jnp.pad(ew, (0, pad))
    onesp = jnp.pad(jnp.ones((E,), jnp.float32), (0, pad))

    degp, cntp = _degree_sc(
        dstp.reshape(NCORE * NSUB, EPT),
        ewp.reshape(NCORE * NSUB, EPT),
        onesp.reshape(NCORE * NSUB, EPT),
    )
    g1, dinv, a_ = _prep(x, degp.T, cntp.T)

    dst3 = dstp.reshape(NSUB, NB, BE)
    ewf = ewp.reshape(NSUB, NBE)
    ar2 = jnp.arange(2, dtype=jnp.int32)
    ar4 = jnp.arange(4, dtype=jnp.int32)
    gidx2 = (srcp[None, :] * 2 + ar2[:, None]).reshape(2 * NSUB, NB, BE)
    gidx4 = (srcp[None, :] * 4 + ar4[:, None]).reshape(4 * NSUB, NB, BE)

    W1r = W1.reshape(2, 128, 512)
    W2r = W2.reshape(4, 128, 512)
    W3r = W3.reshape(4, 128, 256)

    S1 = _agg2(g1.reshape(2 * N, 128), gidx2, dst3, ewf).reshape(2, N, 128)
    g2 = _make_layer(2, 512, True, True, True)(
        S1, g1, a_, W1r, b1.reshape(1, 512), a1.reshape(1, 1), dinv)
    S2 = _agg4(g2.reshape(4 * N, 128), gidx4, dst3, ewf).reshape(4, N, 128)
    h2 = _make_layer(4, 512, True, True, False)(
        S2, g2, a_, W2r, b2.reshape(1, 512), a2.reshape(1, 1))
    g3 = _make_layer(4, 256, False, False, True)(h2, W3r, dinv)
    S3 = _agg2(g3.reshape(2 * N, 128), gidx2, dst3, ewf).reshape(2, N, 128)
    return _final(S3, g3, a_, b3.reshape(1, 256))


# speedup vs baseline: 5.8663x; 5.8663x over previous
import functools

import jax
import jax.numpy as jnp
from jax import lax
from jax.experimental import pallas as pl
from jax.experimental.pallas import tpu as pltpu
from jax.experimental.pallas import tpu_sc as plsc

N = 10000
E = 160000
NSUB = 16
NCORE = 2
BE = 128
NB = 80
EP = NSUB * BE * NB
EPT = EP // (NCORE * NSUB)
NBE = NB * BE
ROWS_PER_TILE = N // NSUB
M_BLK = 2000
M_GRID = N // M_BLK

_mesh = plsc.VectorSubcoreMesh(core_axis_name="c", subcore_axis_name="s")
_sc_params = pltpu.CompilerParams(
    needs_layout_passes=False, use_tc_tiling_on_sc=False)


@functools.partial(
    pl.kernel,
    out_type=(
        jax.ShapeDtypeStruct((NCORE * NSUB * N,), jnp.float32),
        jax.ShapeDtypeStruct((NCORE * NSUB * N,), jnp.float32),
    ),
    mesh=_mesh,
    scratch_types=[
        pltpu.VMEM((EPT,), jnp.int32),
        pltpu.VMEM((EPT,), jnp.float32),
        pltpu.VMEM((EPT,), jnp.float32),
        pltpu.VMEM((N,), jnp.float32),
        pltpu.VMEM((N,), jnp.float32),
    ],
    compiler_params=_sc_params,
)
def _degree_sc(dst1, ew1, ones1, degp, cntp, dst_v, ew_v, on_v, dacc, cacc):
    c = lax.axis_index("c")
    s = lax.axis_index("s")
    wid = s * NCORE + c
    pltpu.sync_copy(dst1.at[pl.ds(wid * EPT, EPT)], dst_v)
    pltpu.sync_copy(ew1.at[pl.ds(wid * EPT, EPT)], ew_v)
    pltpu.sync_copy(ones1.at[pl.ds(wid * EPT, EPT)], on_v)

    @pl.loop(0, N // 16)
    def _zero(i):
        sl = pl.ds(i * 16, 16)
        dacc[sl] = jnp.zeros((16,), jnp.float32)
        cacc[sl] = jnp.zeros((16,), jnp.float32)

    @pl.loop(0, EPT // 16)
    def _edges(b):
        sl = pl.ds(b * 16, 16)
        idx = dst_v[sl]
        plsc.addupdate_scatter(dacc, [idx], ew_v[sl])
        plsc.addupdate_scatter(cacc, [idx], on_v[sl])

    pltpu.sync_copy(dacc, degp.at[pl.ds(wid * N, N)])
    pltpu.sync_copy(cacc, cntp.at[pl.ds(wid * N, N)])


CHW = 64


def _make_agg(nch):
    nch_sc = nch // NCORE

    @functools.partial(
        pl.kernel,
        out_type=jax.ShapeDtypeStruct((nch * N, CHW), jnp.float32),
        mesh=_mesh,
        scratch_types=[
            pltpu.VMEM((NBE,), jnp.int32),
            pltpu.VMEM((NBE,), jnp.int32),
            pltpu.VMEM((NB, BE), jnp.int32),
            pltpu.VMEM((NBE,), jnp.float32),
            pltpu.VMEM((BE, CHW), jnp.float32),
            pltpu.VMEM((BE, CHW), jnp.float32),
            pltpu.VMEM((125, CHW), jnp.float32),
            pltpu.VMEM_SHARED((N, CHW), jnp.float32),
            pltpu.SemaphoreType.DMA,
            pltpu.SemaphoreType.DMA,
            pltpu.SemaphoreType.DMA,
            pltpu.SemaphoreType.DMA,
        ],
        compiler_params=_sc_params,
    )
    def _agg(g2d, srcf, dst3, ewf, s_out, src_v, gidx_v, dst_v, ew_v, gbuf0,
             gbuf1, zbuf, acc, gsem0, gsem1, ssem0, ssem1):
        c = lax.axis_index("c")
        s = lax.axis_index("s")
        pltpu.sync_copy(srcf.at[s], src_v)
        pltpu.sync_copy(dst3.at[s], dst_v)
        pltpu.sync_copy(ewf.at[s], ew_v)

        @pl.loop(0, 125)
        def _zb(r):
            for j in range(CHW // 16):
                zbuf[r, pl.ds(j * 16, 16)] = jnp.zeros((16,), jnp.float32)

        def fire_gather(b, buf, sem):
            idx = gidx_v.at[pl.ds(b * BE, BE)]
            pltpu.async_copy(g2d.at[idx], buf, sem)

        def wait_gather(b, buf, sem):
            idx = gidx_v.at[pl.ds(b * BE, BE)]
            pltpu.make_async_copy(g2d.at[idx], buf, sem).wait()

        def fire_scatter(b, buf, sem):
            pltpu.async_copy(buf, acc.at[dst_v.at[b]], sem, add=True)

        def wait_scatter(b, buf, sem):
            pltpu.make_async_copy(buf, acc.at[dst_v.at[b]], sem).wait()

        def scale(b, buf):
            @pl.loop(0, BE)
            def _scale(e):
                i16 = jnp.full((16,), b * BE + e, dtype=jnp.int32)
                w16 = plsc.load_gather(ew_v, [i16])
                for j in range(CHW // 16):
                    sl = pl.ds(j * 16, 16)
                    buf[e, sl] = buf[e, sl] * w16

        for k in range(nch_sc):
            chunk = c * nch_sc + k
            for p in range(5):
                pltpu.sync_copy(
                    zbuf, acc.at[pl.ds(s * ROWS_PER_TILE + p * 125, 125)])

            @pl.loop(0, NBE // 16)
            def _ix(i):
                sl = pl.ds(i * 16, 16)
                gidx_v[sl] = src_v[sl] * nch + chunk

            fire_gather(0, gbuf0, gsem0)
            plsc.subcore_barrier()

            @pl.loop(0, NB // 2)
            def _pair(p):
                b0 = p * 2
                b1 = b0 + 1

                @pl.when(p > 0)
                def _():
                    wait_scatter(b0 - 1, gbuf1, ssem1)

                fire_gather(b1, gbuf1, gsem1)
                wait_gather(b0, gbuf0, gsem0)
                scale(b0, gbuf0)
                fire_scatter(b0, gbuf0, ssem0)
                wait_gather(b1, gbuf1, gsem1)
                wait_scatter(b0, gbuf0, ssem0)

                @pl.when(b1 + 1 < NB)
                def _():
                    fire_gather(b1 + 1, gbuf0, gsem0)

                scale(b1, gbuf1)
                fire_scatter(b1, gbuf1, ssem1)

            wait_scatter(NB - 1, gbuf1, ssem1)
            plsc.subcore_barrier()
            pltpu.sync_copy(
                acc.at[pl.ds(s * ROWS_PER_TILE, ROWS_PER_TILE)],
                s_out.at[pl.ds(chunk * N + s * ROWS_PER_TILE, ROWS_PER_TILE)])

    return _agg


_agg4 = _make_agg(4)
_agg8 = _make_agg(8)


def _prep_body(x_ref, degp_ref, cntp_ref, g1_ref, dinv_ref, a_ref):
    deg = jnp.sum(degp_ref[...], axis=1, keepdims=True) + 1.0
    cnt = jnp.sum(cntp_ref[...], axis=1, keepdims=True) + 1.0
    dinv = jnp.where(deg > 0, lax.rsqrt(deg), 0.0)
    dinv_ref[...] = dinv
    a_ref[...] = dinv / cnt
    g1_ref[...] = x_ref[...] * dinv


def _prep(x, degpT, cntpT):
    return pl.pallas_call(
        _prep_body,
        grid=(M_GRID,),
        in_specs=[
            pl.BlockSpec((M_BLK, 256), lambda m: (m, 0)),
            pl.BlockSpec((M_BLK, NCORE * NSUB), lambda m: (m, 0)),
            pl.BlockSpec((M_BLK, NCORE * NSUB), lambda m: (m, 0)),
        ],
        out_specs=[
            pl.BlockSpec((M_BLK, 256), lambda m: (m, 0)),
            pl.BlockSpec((M_BLK, 1), lambda m: (m, 0)),
            pl.BlockSpec((M_BLK, 1), lambda m: (m, 0)),
        ],
        out_shape=[
            jax.ShapeDtypeStruct((N, 256), jnp.float32),
            jax.ShapeDtypeStruct((N, 1), jnp.float32),
            jax.ShapeDtypeStruct((N, 1), jnp.float32),
        ],
    )(x, degpT, cntpT)


def _make_layer(nch, d_out, use_s, prelu, scale_out):
    def body(*refs):
        i = 0
        if use_s:
            s_ref, g_ref, a_ref = refs[0], refs[1], refs[2]
            i = 3
        else:
            g_ref = refs[0]
            i = 1
        w_ref = refs[i]
        i += 1
        if prelu:
            b_ref, al_ref = refs[i], refs[i + 1]
            i += 2
        if scale_out:
            dinv_ref = refs[i]
            i += 1
        out_ref = refs[i]
        k = pl.program_id(1)
        if use_s:
            scat = jnp.concatenate([s_ref[0], s_ref[1]], axis=1)
            A = (scat + g_ref[...]) * a_ref[...]
        else:
            A = g_ref[...]
        part = jnp.dot(A, w_ref[0], preferred_element_type=jnp.float32)

        @pl.when(k == 0)
        def _():
            out_ref[...] = part

        @pl.when(k > 0)
        def _():
            out_ref[...] += part

        @pl.when(k == nch - 1)
        def _():
            z = out_ref[...]
            if prelu:
                z = z + b_ref[...]
                z = jnp.where(z >= 0, z, al_ref[0, 0] * z)
            if scale_out:
                z = z * dinv_ref[...]
            out_ref[...] = z

    in_specs = []
    if use_s:
        in_specs.append(pl.BlockSpec((2, M_BLK, CHW), lambda m, k: (k, m, 0)))
        in_specs.append(pl.BlockSpec((M_BLK, 128), lambda m, k: (m, k)))
        in_specs.append(pl.BlockSpec((M_BLK, 1), lambda m, k: (m, 0)))
    else:
        in_specs.append(pl.BlockSpec((M_BLK, 128), lambda m, k: (m, k)))
    in_specs.append(pl.BlockSpec((1, 128, d_out), lambda m, k: (k, 0, 0)))
    if prelu:
        in_specs.append(pl.BlockSpec((1, d_out), lambda m, k: (0, 0)))
        in_specs.append(pl.BlockSpec((1, 1), lambda m, k: (0, 0)))
    if scale_out:
        in_specs.append(pl.BlockSpec((M_BLK, 1), lambda m, k: (m, 0)))

    return pl.pallas_call(
        body,
        grid=(M_GRID, nch),
        in_specs=in_specs,
        out_specs=pl.BlockSpec((M_BLK, d_out), lambda m, k: (m, 0)),
        out_shape=jax.ShapeDtypeStruct((N, d_out), jnp.float32),
    )


def _final_body(s_ref, g_ref, a_ref, b_ref, out_ref):
    scat = jnp.concatenate([s_ref[i] for i in range(4)], axis=1)
    z = (scat + g_ref[...]) * a_ref[...] + b_ref[...]
    out_ref[...] = (jax.nn.sigmoid(z) - 0.5) * 2.0


def _final(s3, g3, a_, b3):
    return pl.pallas_call(
        _final_body,
        grid=(M_GRID,),
        in_specs=[
            pl.BlockSpec((4, M_BLK, CHW), lambda m: (0, m, 0)),
            pl.BlockSpec((M_BLK, 256), lambda m: (m, 0)),
            pl.BlockSpec((M_BLK, 1), lambda m: (m, 0)),
            pl.BlockSpec((1, 256), lambda m: (0, 0)),
        ],
        out_specs=pl.BlockSpec((M_BLK, 256), lambda m: (m, 0)),
        out_shape=jax.ShapeDtypeStruct((N, 256), jnp.float32),
    )(s3, g3, a_, b3)


def kernel(x, edge_index, edge_attr, W1, b1, W2, b2, W3, b3, a1, a2):
    src = edge_index[0].astype(jnp.int32)
    dst = edge_index[1].astype(jnp.int32)
    ew = edge_attr.astype(jnp.float32)
    pad = EP - E
    srcp = jnp.pad(src, (0, pad))
    dstp = jnp.pad(dst, (0, pad))
    ewp = jnp.pad(ew, (0, pad))
    onesp = jnp.pad(jnp.ones((E,), jnp.float32), (0, pad))

    degp, cntp = _degree_sc(dstp, ewp, onesp)
    degpT = degp.reshape(NCORE * NSUB, N).T
    cntpT = cntp.reshape(NCORE * NSUB, N).T
    g1, dinv, a_ = _prep(x, degpT, cntpT)

    srcf = srcp.reshape(NSUB, NBE)
    dst3 = dstp.reshape(NSUB, NB, BE)
    ewf = ewp.reshape(NSUB, NBE)

    W1r = W1.reshape(2, 128, 512)
    W2r = W2.reshape(4, 128, 512)
    W3r = W3.reshape(4, 128, 256)

    S1 = _agg4(g1.reshape(4 * N, CHW), srcf, dst3, ewf).reshape(4, N, CHW)
    g2 = _make_layer(2, 512, True, True, True)(
        S1, g1, a_, W1r, b1.reshape(1, 512), a1.reshape(1, 1), dinv)
    S2 = _agg8(g2.reshape(8 * N, CHW), srcf, dst3, ewf).reshape(8, N, CHW)
    h2 = _make_layer(4, 512, True, True, False)(
        S2, g2, a_, W2r, b2.reshape(1, 512), a2.reshape(1, 1))
    g3 = _make_layer(4, 256, False, False, True)(h2, W3r, dinv)
    S3 = _agg4(g3.reshape(4 * N, CHW), srcf, dst3, ewf).reshape(4, N, CHW)
    return _final(S3, g3, a_, b3.reshape(1, 256))


# speedup vs baseline: 6.7553x; 1.1515x over previous
import functools

import jax
import jax.numpy as jnp
from jax import lax
from jax.experimental import pallas as pl
from jax.experimental.pallas import tpu as pltpu
from jax.experimental.pallas import tpu_sc as plsc

N = 10000
E = 160000
NSUB = 16
NCORE = 2
BE = 64
NB = 160
EP = NSUB * BE * NB
EPT = EP // (NCORE * NSUB)
NBE = NB * BE
ROWS_PER_TILE = N // NSUB
M_BLK = 2000
M_GRID = N // M_BLK

_mesh = plsc.VectorSubcoreMesh(core_axis_name="c", subcore_axis_name="s")
_sc_params = pltpu.CompilerParams(
    needs_layout_passes=False, use_tc_tiling_on_sc=False)


@functools.partial(
    pl.kernel,
    out_type=(
        jax.ShapeDtypeStruct((NCORE * NSUB * N,), jnp.float32),
        jax.ShapeDtypeStruct((NCORE * NSUB * N,), jnp.float32),
    ),
    mesh=_mesh,
    scratch_types=[
        pltpu.VMEM((EPT,), jnp.int32),
        pltpu.VMEM((EPT,), jnp.float32),
        pltpu.VMEM((EPT,), jnp.float32),
        pltpu.VMEM((N,), jnp.float32),
        pltpu.VMEM((N,), jnp.float32),
    ],
    compiler_params=_sc_params,
)
def _degree_sc(dst1, ew1, ones1, degp, cntp, dst_v, ew_v, on_v, dacc, cacc):
    c = lax.axis_index("c")
    s = lax.axis_index("s")
    wid = s * NCORE + c
    pltpu.sync_copy(dst1.at[pl.ds(wid * EPT, EPT)], dst_v)
    pltpu.sync_copy(ew1.at[pl.ds(wid * EPT, EPT)], ew_v)
    pltpu.sync_copy(ones1.at[pl.ds(wid * EPT, EPT)], on_v)

    @pl.loop(0, N // 16)
    def _zero(i):
        sl = pl.ds(i * 16, 16)
        dacc[sl] = jnp.zeros((16,), jnp.float32)
        cacc[sl] = jnp.zeros((16,), jnp.float32)

    @pl.loop(0, EPT // 16)
    def _edges(b):
        sl = pl.ds(b * 16, 16)
        idx = dst_v[sl]
        plsc.addupdate_scatter(dacc, [idx], ew_v[sl])
        plsc.addupdate_scatter(cacc, [idx], on_v[sl])

    pltpu.sync_copy(dacc, degp.at[pl.ds(wid * N, N)])
    pltpu.sync_copy(cacc, cntp.at[pl.ds(wid * N, N)])


CHW = 128


def _make_agg(nch):
    nch_sc = nch // NCORE

    @functools.partial(
        pl.kernel,
        out_type=jax.ShapeDtypeStruct((nch * N, CHW), jnp.float32),
        mesh=_mesh,
        scratch_types=[
            pltpu.VMEM((NBE,), jnp.int32),
            pltpu.VMEM((NB, BE), jnp.int32),
            pltpu.VMEM((NBE,), jnp.float32),
            pltpu.VMEM((BE, CHW), jnp.float32),
            pltpu.VMEM((BE, CHW), jnp.float32),
            pltpu.VMEM_SHARED((N, CHW), jnp.float32),
            pltpu.SemaphoreType.DMA,
            pltpu.SemaphoreType.DMA,
            pltpu.SemaphoreType.DMA,
            pltpu.SemaphoreType.DMA,
        ],
        compiler_params=_sc_params,
    )
    def _agg(g2d, srcf, dst3, ewf, s_out, gidx_v, dst_v, ew_v, gbuf0,
             gbuf1, acc, gsem0, gsem1, ssem0, ssem1):
        c = lax.axis_index("c")
        s = lax.axis_index("s")
        pltpu.sync_copy(dst3.at[s], dst_v)
        pltpu.sync_copy(ewf.at[s], ew_v)

        def fire_gather(b, buf, sem):
            idx = gidx_v.at[pl.ds(b * BE, BE)]
            pltpu.async_copy(g2d.at[idx], buf, sem)

        def wait_gather(b, buf, sem):
            idx = gidx_v.at[pl.ds(b * BE, BE)]
            pltpu.make_async_copy(g2d.at[idx], buf, sem).wait()

        def fire_scatter(b, buf, sem):
            pltpu.async_copy(buf, acc.at[dst_v.at[b]], sem, add=True)

        def wait_scatter(b, buf, sem):
            pltpu.make_async_copy(buf, acc.at[dst_v.at[b]], sem).wait()

        def scale(b, buf):
            @pl.loop(0, BE)
            def _scale(e):
                i16 = jnp.full((16,), b * BE + e, dtype=jnp.int32)
                w16 = plsc.load_gather(ew_v, [i16])
                for j in range(CHW // 16):
                    sl = pl.ds(j * 16, 16)
                    buf[e, sl] = buf[e, sl] * w16

        for k in range(nch_sc):
            chunk = c * nch_sc + k

            @pl.loop(0, BE)
            def _zb(r):
                for j in range(CHW // 16):
                    gbuf0[r, pl.ds(j * 16, 16)] = jnp.zeros((16,), jnp.float32)

            for p in range(ROWS_PER_TILE // BE):
                pltpu.sync_copy(
                    gbuf0, acc.at[pl.ds(s * ROWS_PER_TILE + p * BE, BE)])
            _rem = ROWS_PER_TILE % BE
            if _rem:
                pltpu.sync_copy(
                    gbuf0.at[pl.ds(0, _rem)],
                    acc.at[pl.ds(s * ROWS_PER_TILE + ROWS_PER_TILE - _rem,
                                 _rem)])

            pltpu.sync_copy(srcf.at[s], gidx_v)

            @pl.loop(0, NBE // 16)
            def _ix(i):
                sl = pl.ds(i * 16, 16)
                gidx_v[sl] = gidx_v[sl] * nch + chunk

            fire_gather(0, gbuf0, gsem0)
            plsc.subcore_barrier()

            @pl.loop(0, NB // 2)
            def _pair(p):
                b0 = p * 2
                b1 = b0 + 1

                @pl.when(p > 0)
                def _():
                    wait_scatter(b0 - 1, gbuf1, ssem1)

                fire_gather(b1, gbuf1, gsem1)
                wait_gather(b0, gbuf0, gsem0)
                scale(b0, gbuf0)
                fire_scatter(b0, gbuf0, ssem0)
                wait_gather(b1, gbuf1, gsem1)
                wait_scatter(b0, gbuf0, ssem0)

                @pl.when(b1 + 1 < NB)
                def _():
                    fire_gather(b1 + 1, gbuf0, gsem0)

                scale(b1, gbuf1)
                fire_scatter(b1, gbuf1, ssem1)

            wait_scatter(NB - 1, gbuf1, ssem1)
            plsc.subcore_barrier()
            pltpu.sync_copy(
                acc.at[pl.ds(s * ROWS_PER_TILE, ROWS_PER_TILE)],
                s_out.at[pl.ds(chunk * N + s * ROWS_PER_TILE, ROWS_PER_TILE)])

    return _agg


_agg2 = _make_agg(2)
_agg4 = _make_agg(4)


def _prep_body(x_ref, degp_ref, cntp_ref, g1_ref, dinv_ref, a_ref):
    deg = jnp.sum(degp_ref[...], axis=1, keepdims=True) + 1.0
    cnt = jnp.sum(cntp_ref[...], axis=1, keepdims=True) + 1.0
    dinv = jnp.where(deg > 0, lax.rsqrt(deg), 0.0)
    dinv_ref[...] = dinv
    a_ref[...] = dinv / cnt
    g1_ref[...] = x_ref[...] * dinv


def _prep(x, degpT, cntpT):
    return pl.pallas_call(
        _prep_body,
        grid=(M_GRID,),
        in_specs=[
            pl.BlockSpec((M_BLK, 256), lambda m: (m, 0)),
            pl.BlockSpec((M_BLK, NCORE * NSUB), lambda m: (m, 0)),
            pl.BlockSpec((M_BLK, NCORE * NSUB), lambda m: (m, 0)),
        ],
        out_specs=[
            pl.BlockSpec((M_BLK, 256), lambda m: (m, 0)),
            pl.BlockSpec((M_BLK, 1), lambda m: (m, 0)),
            pl.BlockSpec((M_BLK, 1), lambda m: (m, 0)),
        ],
        out_shape=[
            jax.ShapeDtypeStruct((N, 256), jnp.float32),
            jax.ShapeDtypeStruct((N, 1), jnp.float32),
            jax.ShapeDtypeStruct((N, 1), jnp.float32),
        ],
    )(x, degpT, cntpT)


def _make_layer(nch, d_out, use_s, prelu, scale_out):
    def body(*refs):
        i = 0
        if use_s:
            s_ref, g_ref, a_ref = refs[0], refs[1], refs[2]
            i = 3
        else:
            g_ref = refs[0]
            i = 1
        w_ref = refs[i]
        i += 1
        if prelu:
            b_ref, al_ref = refs[i], refs[i + 1]
            i += 2
        if scale_out:
            dinv_ref = refs[i]
            i += 1
        out_ref = refs[i]
        k = pl.program_id(1)
        if use_s:
            A = (s_ref[0] + g_ref[...]) * a_ref[...]
        else:
            A = g_ref[...]
        part = jnp.dot(A, w_ref[0], preferred_element_type=jnp.float32)

        @pl.when(k == 0)
        def _():
            out_ref[...] = part

        @pl.when(k > 0)
        def _():
            out_ref[...] += part

        @pl.when(k == nch - 1)
        def _():
            z = out_ref[...]
            if prelu:
                z = z + b_ref[...]
                z = jnp.where(z >= 0, z, al_ref[0, 0] * z)
            if scale_out:
                z = z * dinv_ref[...]
            out_ref[...] = z

    in_specs = []
    if use_s:
        in_specs.append(pl.BlockSpec((1, M_BLK, CHW), lambda m, k: (k, m, 0)))
        in_specs.append(pl.BlockSpec((M_BLK, 128), lambda m, k: (m, k)))
        in_specs.append(pl.BlockSpec((M_BLK, 1), lambda m, k: (m, 0)))
    else:
        in_specs.append(pl.BlockSpec((M_BLK, 128), lambda m, k: (m, k)))
    in_specs.append(pl.BlockSpec((1, 128, d_out), lambda m, k: (k, 0, 0)))
    if prelu:
        in_specs.append(pl.BlockSpec((1, d_out), lambda m, k: (0, 0)))
        in_specs.append(pl.BlockSpec((1, 1), lambda m, k: (0, 0)))
    if scale_out:
        in_specs.append(pl.BlockSpec((M_BLK, 1), lambda m, k: (m, 0)))

    return pl.pallas_call(
        body,
        grid=(M_GRID, nch),
        in_specs=in_specs,
        out_specs=pl.BlockSpec((M_BLK, d_out), lambda m, k: (m, 0)),
        out_shape=jax.ShapeDtypeStruct((N, d_out), jnp.float32),
    )


def _final_body(s_ref, g_ref, a_ref, b_ref, out_ref):
    scat = jnp.concatenate([s_ref[0], s_ref[1]], axis=1)
    z = (scat + g_ref[...]) * a_ref[...] + b_ref[...]
    out_ref[...] = (jax.nn.sigmoid(z) - 0.5) * 2.0


def _final(s3, g3, a_, b3):
    return pl.pallas_call(
        _final_body,
        grid=(M_GRID,),
        in_specs=[
            pl.BlockSpec((2, M_BLK, CHW), lambda m: (0, m, 0)),
            pl.BlockSpec((M_BLK, 256), lambda m: (m, 0)),
            pl.BlockSpec((M_BLK, 1), lambda m: (m, 0)),
            pl.BlockSpec((1, 256), lambda m: (0, 0)),
        ],
        out_specs=pl.BlockSpec((M_BLK, 256), lambda m: (m, 0)),
        out_shape=jax.ShapeDtypeStruct((N, 256), jnp.float32),
    )(s3, g3, a_, b3)


def kernel(x, edge_index, edge_attr, W1, b1, W2, b2, W3, b3, a1, a2):
    src = edge_index[0].astype(jnp.int32)
    dst = edge_index[1].astype(jnp.int32)
    ew = edge_attr.astype(jnp.float32)
    pad = EP - E
    srcp = jnp.pad(src, (0, pad))
    dstp = jnp.pad(dst, (0, pad))
    ewp = jnp.pad(ew, (0, pad))
    onesp = jnp.pad(jnp.ones((E,), jnp.float32), (0, pad))

    degp, cntp = _degree_sc(dstp, ewp, onesp)
    degpT = degp.reshape(NCORE * NSUB, N).T
    cntpT = cntp.reshape(NCORE * NSUB, N).T
    g1, dinv, a_ = _prep(x, degpT, cntpT)

    srcf = srcp.reshape(NSUB, NBE)
    dst3 = dstp.reshape(NSUB, NB, BE)
    ewf = ewp.reshape(NSUB, NBE)

    W1r = W1.reshape(2, 128, 512)
    W2r = W2.reshape(4, 128, 512)
    W3r = W3.reshape(4, 128, 256)

    S1 = _agg2(g1.reshape(2 * N, CHW), srcf, dst3, ewf).reshape(2, N, CHW)
    g2 = _make_layer(2, 512, True, True, True)(
        S1, g1, a_, W1r, b1.reshape(1, 512), a1.reshape(1, 1), dinv)
    S2 = _agg4(g2.reshape(4 * N, CHW), srcf, dst3, ewf).reshape(4, N, CHW)
    h2 = _make_layer(4, 512, True, True, False)(
        S2, g2, a_, W2r, b2.reshape(1, 512), a2.reshape(1, 1))
    g3 = _make_layer(4, 256, False, False, True)(h2, W3r, dinv)
    S3 = _agg2(g3.reshape(2 * N, CHW), srcf, dst3, ewf).reshape(2, N, CHW)
    return _final(S3, g3, a_, b3.reshape(1, 256))
